# baseline (speedup 1.0000x reference)
import jax
import jax.numpy as jnp
from jax import lax
from jax.experimental import pallas as pl
from jax.experimental.pallas import tpu as pltpu
from jax.experimental.pallas import tpu_sc as plsc

S = 2048
D = 1024
E = 8
DFF = 4096
CAP = 512

NC = 2
NS = 16
NW = NC * NS
SLOTS = E * CAP
SPW = SLOTS // NW
RPT = 64

EPC = E // 2
FCH = 4
DFC = DFF // FCH

_INTERPRET = False


def _gating_kernel(hs_ref, wg_ref, route_ref, laux_ref, cnt_ref):
    hs = hs_ref[...]
    wg = wg_ref[...]
    logits = jnp.dot(hs, wg, preferred_element_type=jnp.float32)
    gates = jax.nn.softmax(logits, axis=-1)
    iota_e = jax.lax.broadcasted_iota(jnp.int32, (S, E), 1)
    gmax = jnp.max(gates, axis=1, keepdims=True)
    idx1 = jnp.min(jnp.where(gates == gmax, iota_e, E), axis=1, keepdims=True)
    mask1 = iota_e == idx1
    lx = jnp.where(mask1, -jnp.inf, logits)
    lmax = jnp.max(lx, axis=1, keepdims=True)
    idx2 = jnp.min(jnp.where(lx == lmax, iota_e, E), axis=1, keepdims=True)
    mask2 = iota_e == idx2
    m1f = mask1.astype(jnp.float32)
    m2f = mask2.astype(jnp.float32)
    ir = jax.lax.broadcasted_iota(jnp.int32, (S, S), 0)
    ic = jax.lax.broadcasted_iota(jnp.int32, (S, S), 1)
    ltri = (ic < ir).astype(jnp.float32)
    loc1 = jnp.dot(ltri, m1f, preferred_element_type=jnp.float32)
    count1 = jnp.sum(m1f, axis=0, keepdims=True)
    loc2 = jnp.dot(ltri, m2f, preferred_element_type=jnp.float32) + count1
    me = jnp.mean(gates, axis=0, keepdims=True)
    ce = jnp.mean(m1f, axis=0, keepdims=True)
    laux_ref[...] = jnp.sum(me * ce, axis=1, keepdims=True) * jnp.float32(E)
    cnt_ref[...] = count1.astype(jnp.int32)
    c1 = jnp.sum(jnp.where(mask1, loc1, 0.0), axis=1, keepdims=True)
    c2 = jnp.sum(jnp.where(mask2, loc2, 0.0), axis=1, keepdims=True)
    k1 = c1 < CAP
    k2 = c2 < CAP
    g1 = jnp.where(k1, jnp.sum(jnp.where(mask1, gates, 0.0), axis=1, keepdims=True), 0.0)
    g2 = jnp.where(k2, jnp.sum(jnp.where(mask2, gates, 0.0), axis=1, keepdims=True), 0.0)
    denom = jnp.maximum(g1 + g2, jnp.finfo(jnp.float32).eps)
    g1 = g1 / denom
    g2 = g2 / denom
    route = jnp.concatenate(
        [idx1.astype(jnp.float32), idx2.astype(jnp.float32), c1, c2, g1, g2,
         jnp.zeros((S, 2), jnp.float32)], axis=1)
    route_ref[...] = route


def _slotmap_body(route_hbm, src_hbm, rv, src_v):
    wid = lax.axis_index("s") * NC + lax.axis_index("c")

    @pl.when(wid == 0)
    def _():
        pltpu.sync_copy(route_hbm, rv)

        def zero_body(i, carry):
            src_v[pl.ds(i * 16, 16)] = jnp.zeros((16,), jnp.int32)
            return carry

        lax.fori_loop(0, SLOTS // 16, zero_body, 0)

        def scat_body(i, carry):
            tok = lax.iota(jnp.int32, 16) + jnp.full((16,), 16, jnp.int32) * i
            a = tok * jnp.full((16,), 8, jnp.int32)
            idx1 = plsc.load_gather(rv, [a]).astype(jnp.int32)
            idx2 = plsc.load_gather(rv, [a + 1]).astype(jnp.int32)
            c1 = plsc.load_gather(rv, [a + 2]).astype(jnp.int32)
            c2 = plsc.load_gather(rv, [a + 3]).astype(jnp.int32)
            k1 = c1 < CAP
            k2 = c2 < CAP
            s1 = jnp.where(k1, idx1 * CAP + c1, 0)
            s2 = jnp.where(k2, idx2 * CAP + c2, 0)
            plsc.store_scatter(src_v, [s1], tok, mask=k1)
            plsc.store_scatter(src_v, [s2], tok, mask=k2)
            return carry

        lax.fori_loop(0, S // 16, scat_body, 0)
        pltpu.sync_copy(src_v, src_hbm)


def _dispatch_body(src_hbm, hs_hbm, disp_hbm, idx_v, rows_v, sem):
    wid = lax.axis_index("s") * NC + lax.axis_index("c")
    base = wid * SPW
    for r in range(SPW // RPT):
        off = base + r * RPT
        pltpu.sync_copy(src_hbm.at[pl.ds(off, RPT)], idx_v)
        pltpu.async_copy(hs_hbm.at[idx_v], rows_v, sem).wait()
        pltpu.sync_copy(rows_v, disp_hbm.at[pl.ds(off, RPT)])


def _ffn_kernel(disp_ref, w1_ref, b1_ref, w2_ref, b2_ref,
                eo_ref, db_ref, acc_ref):
    ff = pl.program_id(2)

    @pl.when(ff == 0)
    def _():
        db_ref[...] = disp_ref[...].astype(jnp.bfloat16)
        acc_ref[...] = jnp.broadcast_to(b2_ref[0, 0][None, :], (CAP, D))

    h = jnp.maximum(
        jnp.dot(db_ref[...], w1_ref[0].astype(jnp.bfloat16),
                preferred_element_type=jnp.float32)
        + b1_ref[0, 0][None, :], 0.0).astype(jnp.bfloat16)
    acc_ref[...] += jnp.dot(h, w2_ref[0].astype(jnp.bfloat16),
                            preferred_element_type=jnp.float32)

    @pl.when(ff == FCH - 1)
    def _():
        eo_ref[0] = acc_ref[...]


SCH = SLOTS // 2 // 1024


def _combine_kernel(route_ref, eo_ref, out_ref):
    c = pl.program_id(0)
    k = pl.program_id(1)
    idx1 = route_ref[:, 0:1].astype(jnp.int32)
    idx2 = route_ref[:, 1:2].astype(jnp.int32)
    c1 = route_ref[:, 2:3].astype(jnp.int32)
    c2 = route_ref[:, 3:4].astype(jnp.int32)
    gs1 = jnp.where(c1 < CAP, idx1 * CAP + c1, -1)
    gs2 = jnp.where(c2 < CAP, idx2 * CAP + c2, -1)
    base = (c * SCH + k) * 1024
    giota = jax.lax.broadcasted_iota(jnp.int32, (S, 1024), 1) + base
    cmb_f = (jnp.where(gs1 == giota, route_ref[:, 4:5], 0.0)
             + jnp.where(gs2 == giota, route_ref[:, 5:6], 0.0))
    contrib = jnp.dot(cmb_f.astype(jnp.bfloat16),
                      eo_ref[...].astype(jnp.bfloat16),
                      preferred_element_type=jnp.float32)

    @pl.when(k == 0)
    def _():
        out_ref[0] = contrib

    @pl.when(k != 0)
    def _():
        out_ref[0] += contrib


def kernel(hidden_states, wg, w1, b1, w2, b2):
    route, laux, counts = pl.pallas_call(
        _gating_kernel,
        out_shape=(
            jax.ShapeDtypeStruct((S, 8), jnp.float32),
            jax.ShapeDtypeStruct((1, 1), jnp.float32),
            jax.ShapeDtypeStruct((1, E), jnp.int32),
        ),
        interpret=_INTERPRET,
    )(hidden_states, wg)

    mesh = plsc.VectorSubcoreMesh(core_axis_name="c", subcore_axis_name="s")

    slot_src = pl.kernel(
        _slotmap_body,
        out_type=jax.ShapeDtypeStruct((SLOTS,), jnp.int32),
        mesh=mesh,
        compiler_params=pltpu.CompilerParams(needs_layout_passes=False),
        scratch_types=[
            pltpu.VMEM((S * 8,), jnp.float32),
            pltpu.VMEM((SLOTS,), jnp.int32),
        ],
    )(route.reshape(S * 8))

    dispatched = pl.kernel(
        _dispatch_body,
        out_type=jax.ShapeDtypeStruct((SLOTS, D), jnp.float32),
        mesh=mesh,
        scratch_types=[
            pltpu.VMEM((RPT,), jnp.int32),
            pltpu.VMEM((RPT, D), jnp.float32),
            pltpu.SemaphoreType.DMA,
        ],
    )(slot_src, hidden_states)

    eo = pl.pallas_call(
        _ffn_kernel,
        grid=(2, EPC, FCH),
        in_specs=[
            pl.BlockSpec((CAP, D), lambda c, ei, ff: (c * EPC + ei, 0)),
            pl.BlockSpec((1, D, DFC), lambda c, ei, ff: (c * EPC + ei, 0, ff)),
            pl.BlockSpec((1, 1, DFC), lambda c, ei, ff: (c * EPC + ei, 0, ff)),
            pl.BlockSpec((1, DFC, D), lambda c, ei, ff: (c * EPC + ei, ff, 0)),
            pl.BlockSpec((1, 1, D), lambda c, ei, ff: (c * EPC + ei, 0, 0)),
        ],
        out_specs=pl.BlockSpec((1, CAP, D), lambda c, ei, ff: (c * EPC + ei, 0, 0)),
        out_shape=jax.ShapeDtypeStruct((E, CAP, D), jnp.float32),
        scratch_shapes=[
            pltpu.VMEM((CAP, D), jnp.bfloat16),
            pltpu.VMEM((CAP, D), jnp.float32),
        ],
        compiler_params=pltpu.CompilerParams(
            dimension_semantics=("parallel", "arbitrary", "arbitrary"),
        ),
        interpret=_INTERPRET,
    )(dispatched, w1, b1.reshape(E, 1, DFF), w2, b2.reshape(E, 1, D))

    out2 = pl.pallas_call(
        _combine_kernel,
        grid=(2, SCH),
        in_specs=[
            pl.BlockSpec((S, 8), lambda c, k: (0, 0)),
            pl.BlockSpec((1024, D), lambda c, k: (c * SCH + k, 0)),
        ],
        out_specs=pl.BlockSpec((1, S, D), lambda c, k: (c, 0, 0)),
        out_shape=jax.ShapeDtypeStruct((2, S, D), jnp.float32),
        compiler_params=pltpu.CompilerParams(
            dimension_semantics=("parallel", "arbitrary"),
        ),
        interpret=_INTERPRET,
    )(route, eo.reshape(SLOTS, D))

    return out2[0] + out2[1], laux.reshape(()), counts.reshape((E,))


# speedup vs baseline: 1.0026x; 1.0026x over previous
import jax
import jax.numpy as jnp
from jax import lax
from jax.experimental import pallas as pl
from jax.experimental.pallas import tpu as pltpu
from jax.experimental.pallas import tpu_sc as plsc

S = 2048
D = 1024
E = 8
DFF = 4096
CAP = 512

NC = 2
NS = 16
NW = NC * NS
SLOTS = E * CAP
SPW = SLOTS // NW
RPT = 64

EPC = E // 2
FCH = 4
DFC = DFF // FCH

_INTERPRET = False


def _gating_kernel(hs_ref, wg_ref, route_ref, laux_ref, cnt_ref):
    hs = hs_ref[...]
    wg = wg_ref[...]
    logits = jnp.dot(hs, wg, preferred_element_type=jnp.float32)
    gates = jax.nn.softmax(logits, axis=-1)
    iota_e = jax.lax.broadcasted_iota(jnp.int32, (S, E), 1)
    gmax = jnp.max(gates, axis=1, keepdims=True)
    idx1 = jnp.min(jnp.where(gates == gmax, iota_e, E), axis=1, keepdims=True)
    mask1 = iota_e == idx1
    lx = jnp.where(mask1, -jnp.inf, logits)
    lmax = jnp.max(lx, axis=1, keepdims=True)
    idx2 = jnp.min(jnp.where(lx == lmax, iota_e, E), axis=1, keepdims=True)
    mask2 = iota_e == idx2
    m1f = mask1.astype(jnp.float32)
    m2f = mask2.astype(jnp.float32)
    ir = jax.lax.broadcasted_iota(jnp.int32, (S, S), 0)
    ic = jax.lax.broadcasted_iota(jnp.int32, (S, S), 1)
    ltri = (ic < ir).astype(jnp.float32)
    loc1 = jnp.dot(ltri, m1f, preferred_element_type=jnp.float32)
    count1 = jnp.sum(m1f, axis=0, keepdims=True)
    loc2 = jnp.dot(ltri, m2f, preferred_element_type=jnp.float32) + count1
    me = jnp.mean(gates, axis=0, keepdims=True)
    ce = jnp.mean(m1f, axis=0, keepdims=True)
    laux_ref[...] = jnp.sum(me * ce, axis=1, keepdims=True) * jnp.float32(E)
    cnt_ref[...] = count1.astype(jnp.int32)
    c1 = jnp.sum(jnp.where(mask1, loc1, 0.0), axis=1, keepdims=True)
    c2 = jnp.sum(jnp.where(mask2, loc2, 0.0), axis=1, keepdims=True)
    k1 = c1 < CAP
    k2 = c2 < CAP
    g1 = jnp.where(k1, jnp.sum(jnp.where(mask1, gates, 0.0), axis=1, keepdims=True), 0.0)
    g2 = jnp.where(k2, jnp.sum(jnp.where(mask2, gates, 0.0), axis=1, keepdims=True), 0.0)
    denom = jnp.maximum(g1 + g2, jnp.finfo(jnp.float32).eps)
    g1 = g1 / denom
    g2 = g2 / denom
    route = jnp.concatenate(
        [idx1.astype(jnp.float32), idx2.astype(jnp.float32), c1, c2, g1, g2,
         jnp.zeros((S, 2), jnp.float32)], axis=1)
    route_ref[...] = route


def _slotmap_body(route_hbm, src_hbm, rv, src_v):
    wid = lax.axis_index("s") * NC + lax.axis_index("c")

    @pl.when(wid == 0)
    def _():
        pltpu.sync_copy(route_hbm, rv)

        def zero_body(i, carry):
            src_v[pl.ds(i * 16, 16)] = jnp.zeros((16,), jnp.int32)
            return carry

        lax.fori_loop(0, SLOTS // 16, zero_body, 0)

        def scat_body(i, carry):
            tok = lax.iota(jnp.int32, 16) + jnp.full((16,), 16, jnp.int32) * i
            a = tok * jnp.full((16,), 8, jnp.int32)
            idx1 = plsc.load_gather(rv, [a]).astype(jnp.int32)
            idx2 = plsc.load_gather(rv, [a + 1]).astype(jnp.int32)
            c1 = plsc.load_gather(rv, [a + 2]).astype(jnp.int32)
            c2 = plsc.load_gather(rv, [a + 3]).astype(jnp.int32)
            k1 = c1 < CAP
            k2 = c2 < CAP
            s1 = jnp.where(k1, idx1 * CAP + c1, 0)
            s2 = jnp.where(k2, idx2 * CAP + c2, 0)
            plsc.store_scatter(src_v, [s1], tok, mask=k1)
            plsc.store_scatter(src_v, [s2], tok, mask=k2)
            return carry

        lax.fori_loop(0, S // 16, scat_body, 0)
        pltpu.sync_copy(src_v, src_hbm)


def _dispatch_body(src_hbm, hs_hbm, disp_hbm, idx_v, rows_v, sem):
    wid = lax.axis_index("s") * NC + lax.axis_index("c")
    base = wid * SPW
    for r in range(SPW // RPT):
        off = base + r * RPT
        pltpu.sync_copy(src_hbm.at[pl.ds(off, RPT)], idx_v)
        pltpu.async_copy(hs_hbm.at[idx_v], rows_v, sem).wait()
        pltpu.sync_copy(rows_v, disp_hbm.at[pl.ds(off, RPT)])


def _ffn_kernel(disp_ref, w1_ref, b1_ref, w2_ref, b2_ref,
                eo_ref, db_ref, acc_ref):
    ff = pl.program_id(2)

    @pl.when(ff == 0)
    def _():
        db_ref[...] = disp_ref[...].astype(jnp.bfloat16)
        acc_ref[...] = jnp.broadcast_to(b2_ref[0, 0][None, :], (CAP, D))

    h = jnp.maximum(
        jnp.dot(db_ref[...], w1_ref[0].astype(jnp.bfloat16),
                preferred_element_type=jnp.float32)
        + b1_ref[0, 0][None, :], 0.0).astype(jnp.bfloat16)
    acc_ref[...] += jnp.dot(h, w2_ref[0].astype(jnp.bfloat16),
                            preferred_element_type=jnp.float32)

    @pl.when(ff == FCH - 1)
    def _():
        eo_ref[0] = acc_ref[...]


SCH = SLOTS // 2 // 1024


def _combine_kernel(route_ref, eo_ref, out_ref):
    c = pl.program_id(0)
    k = pl.program_id(1)
    idx1 = route_ref[:, 0:1].astype(jnp.int32)
    idx2 = route_ref[:, 1:2].astype(jnp.int32)
    c1 = route_ref[:, 2:3].astype(jnp.int32)
    c2 = route_ref[:, 3:4].astype(jnp.int32)
    gs1 = jnp.where(c1 < CAP, idx1 * CAP + c1, -1)
    gs2 = jnp.where(c2 < CAP, idx2 * CAP + c2, -1)
    base = (c * SCH + k) * 1024
    giota = jax.lax.broadcasted_iota(jnp.int32, (S, 1024), 1) + base
    cmb_f = (jnp.where(gs1 == giota, route_ref[:, 4:5], 0.0)
             + jnp.where(gs2 == giota, route_ref[:, 5:6], 0.0))
    contrib = jnp.dot(cmb_f.astype(jnp.bfloat16),
                      eo_ref[...].astype(jnp.bfloat16),
                      preferred_element_type=jnp.float32)

    @pl.when(k == 0)
    def _():
        out_ref[0] = contrib

    @pl.when(k != 0)
    def _():
        out_ref[0] += contrib


def kernel(hidden_states, wg, w1, b1, w2, b2):
    route, laux, counts = pl.pallas_call(
        _gating_kernel,
        out_shape=(
            jax.ShapeDtypeStruct((S, 8), jnp.float32),
            jax.ShapeDtypeStruct((1, 1), jnp.float32),
            jax.ShapeDtypeStruct((1, E), jnp.int32),
        ),
        interpret=_INTERPRET,
    )(hidden_states, wg)

    mesh = plsc.VectorSubcoreMesh(core_axis_name="c", subcore_axis_name="s")

    slot_src = pl.kernel(
        _slotmap_body,
        out_type=jax.ShapeDtypeStruct((SLOTS,), jnp.int32),
        mesh=mesh,
        compiler_params=pltpu.CompilerParams(needs_layout_passes=False),
        scratch_types=[
            pltpu.VMEM((S * 8,), jnp.float32),
            pltpu.VMEM((SLOTS,), jnp.int32),
        ],
    )(route.reshape(S * 8))

    dispatched = pl.kernel(
        _dispatch_body,
        out_type=jax.ShapeDtypeStruct((SLOTS, D), jnp.float32),
        mesh=mesh,
        scratch_types=[
            pltpu.VMEM((RPT,), jnp.int32),
            pltpu.VMEM((RPT, D), jnp.float32),
            pltpu.SemaphoreType.DMA,
        ],
    )(slot_src, hidden_states)

    eo = pl.pallas_call(
        _ffn_kernel,
        grid=(1, E, FCH),
        in_specs=[
            pl.BlockSpec((CAP, D), lambda c, ei, ff: (c * EPC + ei, 0)),
            pl.BlockSpec((1, D, DFC), lambda c, ei, ff: (c * EPC + ei, 0, ff)),
            pl.BlockSpec((1, 1, DFC), lambda c, ei, ff: (c * EPC + ei, 0, ff)),
            pl.BlockSpec((1, DFC, D), lambda c, ei, ff: (c * EPC + ei, ff, 0)),
            pl.BlockSpec((1, 1, D), lambda c, ei, ff: (c * EPC + ei, 0, 0)),
        ],
        out_specs=pl.BlockSpec((1, CAP, D), lambda c, ei, ff: (c * EPC + ei, 0, 0)),
        out_shape=jax.ShapeDtypeStruct((E, CAP, D), jnp.float32),
        scratch_shapes=[
            pltpu.VMEM((CAP, D), jnp.bfloat16),
            pltpu.VMEM((CAP, D), jnp.float32),
        ],
        compiler_params=pltpu.CompilerParams(
            dimension_semantics=("parallel", "arbitrary", "arbitrary"),
        ),
        interpret=_INTERPRET,
    )(dispatched, w1, b1.reshape(E, 1, DFF), w2, b2.reshape(E, 1, D))

    out2 = pl.pallas_call(
        _combine_kernel,
        grid=(1, SLOTS // 1024),
        in_specs=[
            pl.BlockSpec((S, 8), lambda c, k: (0, 0)),
            pl.BlockSpec((1024, D), lambda c, k: (c * SCH + k, 0)),
        ],
        out_specs=pl.BlockSpec((1, S, D), lambda c, k: (c, 0, 0)),
        out_shape=jax.ShapeDtypeStruct((2, S, D), jnp.float32),
        compiler_params=pltpu.CompilerParams(
            dimension_semantics=("parallel", "arbitrary"),
        ),
        interpret=_INTERPRET,
    )(route, eo.reshape(SLOTS, D))

    return out2[0] + out2[1], laux.reshape(()), counts.reshape((E,))


# speedup vs baseline: 1.0321x; 1.0295x over previous
import jax
import jax.numpy as jnp
from jax import lax
from jax.experimental import pallas as pl
from jax.experimental.pallas import tpu as pltpu
from jax.experimental.pallas import tpu_sc as plsc

S = 2048
D = 1024
E = 8
DFF = 4096
CAP = 512

NC = 2
NS = 16
NW = NC * NS
SLOTS = E * CAP
SPW = SLOTS // NW
RPT = 64

EPC = E // 2
FCH = 2
DFC = DFF // FCH

_INTERPRET = False


def _gating_kernel(hs_ref, wg_ref, route_ref, laux_ref, cnt_ref):
    hs = hs_ref[...]
    wg = wg_ref[...]
    logits = jnp.dot(hs, wg, preferred_element_type=jnp.float32)
    gates = jax.nn.softmax(logits, axis=-1)
    iota_e = jax.lax.broadcasted_iota(jnp.int32, (S, E), 1)
    gmax = jnp.max(gates, axis=1, keepdims=True)
    idx1 = jnp.min(jnp.where(gates == gmax, iota_e, E), axis=1, keepdims=True)
    mask1 = iota_e == idx1
    lx = jnp.where(mask1, -jnp.inf, logits)
    lmax = jnp.max(lx, axis=1, keepdims=True)
    idx2 = jnp.min(jnp.where(lx == lmax, iota_e, E), axis=1, keepdims=True)
    mask2 = iota_e == idx2
    m1f = mask1.astype(jnp.float32)
    m2f = mask2.astype(jnp.float32)
    ir = jax.lax.broadcasted_iota(jnp.int32, (S, S), 0)
    ic = jax.lax.broadcasted_iota(jnp.int32, (S, S), 1)
    ltri = (ic < ir).astype(jnp.float32)
    loc1 = jnp.dot(ltri, m1f, preferred_element_type=jnp.float32)
    count1 = jnp.sum(m1f, axis=0, keepdims=True)
    loc2 = jnp.dot(ltri, m2f, preferred_element_type=jnp.float32) + count1
    me = jnp.mean(gates, axis=0, keepdims=True)
    ce = jnp.mean(m1f, axis=0, keepdims=True)
    laux_ref[...] = jnp.sum(me * ce, axis=1, keepdims=True) * jnp.float32(E)
    cnt_ref[...] = count1.astype(jnp.int32)
    c1 = jnp.sum(jnp.where(mask1, loc1, 0.0), axis=1, keepdims=True)
    c2 = jnp.sum(jnp.where(mask2, loc2, 0.0), axis=1, keepdims=True)
    k1 = c1 < CAP
    k2 = c2 < CAP
    g1 = jnp.where(k1, jnp.sum(jnp.where(mask1, gates, 0.0), axis=1, keepdims=True), 0.0)
    g2 = jnp.where(k2, jnp.sum(jnp.where(mask2, gates, 0.0), axis=1, keepdims=True), 0.0)
    denom = jnp.maximum(g1 + g2, jnp.finfo(jnp.float32).eps)
    g1 = g1 / denom
    g2 = g2 / denom
    route = jnp.concatenate(
        [idx1.astype(jnp.float32), idx2.astype(jnp.float32), c1, c2, g1, g2,
         jnp.zeros((S, 2), jnp.float32)], axis=1)
    route_ref[...] = route


def _slotmap_body(route_hbm, src_hbm, rv, src_v):
    wid = lax.axis_index("s") * NC + lax.axis_index("c")

    @pl.when(wid == 0)
    def _():
        pltpu.sync_copy(route_hbm, rv)

        def zero_body(i, carry):
            src_v[pl.ds(i * 16, 16)] = jnp.zeros((16,), jnp.int32)
            return carry

        lax.fori_loop(0, SLOTS // 16, zero_body, 0)

        def scat_body(i, carry):
            tok = lax.iota(jnp.int32, 16) + jnp.full((16,), 16, jnp.int32) * i
            a = tok * jnp.full((16,), 8, jnp.int32)
            idx1 = plsc.load_gather(rv, [a]).astype(jnp.int32)
            idx2 = plsc.load_gather(rv, [a + 1]).astype(jnp.int32)
            c1 = plsc.load_gather(rv, [a + 2]).astype(jnp.int32)
            c2 = plsc.load_gather(rv, [a + 3]).astype(jnp.int32)
            k1 = c1 < CAP
            k2 = c2 < CAP
            s1 = jnp.where(k1, idx1 * CAP + c1, 0)
            s2 = jnp.where(k2, idx2 * CAP + c2, 0)
            plsc.store_scatter(src_v, [s1], tok, mask=k1)
            plsc.store_scatter(src_v, [s2], tok, mask=k2)
            return carry

        lax.fori_loop(0, S // 16, scat_body, 0)
        pltpu.sync_copy(src_v, src_hbm)


def _dispatch_body(src_hbm, hs_hbm, disp_hbm, idx_v, rows_v, sem):
    wid = lax.axis_index("s") * NC + lax.axis_index("c")
    base = wid * SPW
    for r in range(SPW // RPT):
        off = base + r * RPT
        pltpu.sync_copy(src_hbm.at[pl.ds(off, RPT)], idx_v)
        pltpu.async_copy(hs_hbm.at[idx_v], rows_v, sem).wait()
        pltpu.sync_copy(rows_v, disp_hbm.at[pl.ds(off, RPT)])


def _ffn_kernel(disp_ref, w1_ref, b1_ref, w2_ref, b2_ref,
                eo_ref, db_ref, acc_ref):
    ff = pl.program_id(2)

    @pl.when(ff == 0)
    def _():
        db_ref[...] = disp_ref[...].astype(jnp.bfloat16)
        acc_ref[...] = jnp.broadcast_to(b2_ref[0, 0][None, :], (CAP, D))

    h = jnp.maximum(
        jnp.dot(db_ref[...], w1_ref[0].astype(jnp.bfloat16),
                preferred_element_type=jnp.float32)
        + b1_ref[0, 0][None, :], 0.0).astype(jnp.bfloat16)
    acc_ref[...] += jnp.dot(h, w2_ref[0].astype(jnp.bfloat16),
                            preferred_element_type=jnp.float32)

    @pl.when(ff == FCH - 1)
    def _():
        eo_ref[0] = acc_ref[...]


SCH = SLOTS // 2 // 1024


def _combine_kernel(route_ref, eo_ref, out_ref):
    c = pl.program_id(0)
    k = pl.program_id(1)
    idx1 = route_ref[:, 0:1].astype(jnp.int32)
    idx2 = route_ref[:, 1:2].astype(jnp.int32)
    c1 = route_ref[:, 2:3].astype(jnp.int32)
    c2 = route_ref[:, 3:4].astype(jnp.int32)
    gs1 = jnp.where(c1 < CAP, idx1 * CAP + c1, -1)
    gs2 = jnp.where(c2 < CAP, idx2 * CAP + c2, -1)
    base = (c * SCH + k) * 1024
    giota = jax.lax.broadcasted_iota(jnp.int32, (S, 1024), 1) + base
    cmb_f = (jnp.where(gs1 == giota, route_ref[:, 4:5], 0.0)
             + jnp.where(gs2 == giota, route_ref[:, 5:6], 0.0))
    contrib = jnp.dot(cmb_f.astype(jnp.bfloat16),
                      eo_ref[...].astype(jnp.bfloat16),
                      preferred_element_type=jnp.float32)

    @pl.when(k == 0)
    def _():
        out_ref[0] = contrib

    @pl.when(k != 0)
    def _():
        out_ref[0] += contrib


def kernel(hidden_states, wg, w1, b1, w2, b2):
    route, laux, counts = pl.pallas_call(
        _gating_kernel,
        out_shape=(
            jax.ShapeDtypeStruct((S, 8), jnp.float32),
            jax.ShapeDtypeStruct((1, 1), jnp.float32),
            jax.ShapeDtypeStruct((1, E), jnp.int32),
        ),
        interpret=_INTERPRET,
    )(hidden_states, wg)

    mesh = plsc.VectorSubcoreMesh(core_axis_name="c", subcore_axis_name="s")

    slot_src = pl.kernel(
        _slotmap_body,
        out_type=jax.ShapeDtypeStruct((SLOTS,), jnp.int32),
        mesh=mesh,
        compiler_params=pltpu.CompilerParams(needs_layout_passes=False),
        scratch_types=[
            pltpu.VMEM((S * 8,), jnp.float32),
            pltpu.VMEM((SLOTS,), jnp.int32),
        ],
    )(route.reshape(S * 8))

    dispatched = pl.kernel(
        _dispatch_body,
        out_type=jax.ShapeDtypeStruct((SLOTS, D), jnp.float32),
        mesh=mesh,
        scratch_types=[
            pltpu.VMEM((RPT,), jnp.int32),
            pltpu.VMEM((RPT, D), jnp.float32),
            pltpu.SemaphoreType.DMA,
        ],
    )(slot_src, hidden_states)

    eo = pl.pallas_call(
        _ffn_kernel,
        grid=(1, E, FCH),
        in_specs=[
            pl.BlockSpec((CAP, D), lambda c, ei, ff: (c * EPC + ei, 0)),
            pl.BlockSpec((1, D, DFC), lambda c, ei, ff: (c * EPC + ei, 0, ff)),
            pl.BlockSpec((1, 1, DFC), lambda c, ei, ff: (c * EPC + ei, 0, ff)),
            pl.BlockSpec((1, DFC, D), lambda c, ei, ff: (c * EPC + ei, ff, 0)),
            pl.BlockSpec((1, 1, D), lambda c, ei, ff: (c * EPC + ei, 0, 0)),
        ],
        out_specs=pl.BlockSpec((1, CAP, D), lambda c, ei, ff: (c * EPC + ei, 0, 0)),
        out_shape=jax.ShapeDtypeStruct((E, CAP, D), jnp.float32),
        scratch_shapes=[
            pltpu.VMEM((CAP, D), jnp.bfloat16),
            pltpu.VMEM((CAP, D), jnp.float32),
        ],
        compiler_params=pltpu.CompilerParams(
            dimension_semantics=("parallel", "arbitrary", "arbitrary"),
        ),
        interpret=_INTERPRET,
    )(dispatched, w1, b1.reshape(E, 1, DFF), w2, b2.reshape(E, 1, D))

    out2 = pl.pallas_call(
        _combine_kernel,
        grid=(1, SLOTS // 1024),
        in_specs=[
            pl.BlockSpec((S, 8), lambda c, k: (0, 0)),
            pl.BlockSpec((1024, D), lambda c, k: (c * SCH + k, 0)),
        ],
        out_specs=pl.BlockSpec((1, S, D), lambda c, k: (c, 0, 0)),
        out_shape=jax.ShapeDtypeStruct((2, S, D), jnp.float32),
        compiler_params=pltpu.CompilerParams(
            dimension_semantics=("parallel", "arbitrary"),
        ),
        interpret=_INTERPRET,
    )(route, eo.reshape(SLOTS, D))

    return out2[0] + out2[1], laux.reshape(()), counts.reshape((E,))
